# Initial kernel scaffold; baseline (speedup 1.0000x reference)
import jax
import jax.numpy as jnp
from jax.experimental import pallas as pl


def kernel(x, band1, b1t, band2, b2t, wfc1_big, bfc1_r, wfc2_p, bfc2_p):
    raise NotImplementedError("write your pallas kernel here")


# speedup vs baseline: 1.1751x; 1.1751x over previous
import jax
import jax.numpy as jnp
from jax.experimental import pallas as pl
from jax.experimental.pallas import tpu as pltpu

_NUM_CLASSES = 10
_VMEM_LIMIT_BYTES = 48 * 1024 * 1024


def _conv_kernel(x_ref, w1b_ref, b1_ref, w2b_ref, b2_ref, out_ref,
                 xpad_ref, y1pad_ref):
    tb = x_ref.shape[0]

    xpad_ref[...] = jnp.zeros_like(xpad_ref)
    xpad_ref[:, 1:29, :] = x_ref[...]

    acc1 = jnp.zeros((tb * 28, 896), jnp.float32)
    for ky in range(3):
        lhs = xpad_ref[:, ky:ky + 28, :].reshape(tb * 28, 28)
        acc1 = acc1 + jnp.dot(lhs, w1b_ref[ky],
                              preferred_element_type=jnp.float32)
    y1 = jnp.maximum(acc1 + b1_ref[...], 0.0)

    y1pad_ref[...] = jnp.zeros_like(y1pad_ref)
    y1pad_ref[:, 1:29, :] = y1.reshape(tb, 28, 896).astype(jnp.bfloat16)

    acc2 = jnp.zeros((tb * 28, 896), jnp.float32)
    for ky in range(3):
        lhs = y1pad_ref[:, ky:ky + 28, :].reshape(tb * 28, 896)
        acc2 = acc2 + jnp.dot(lhs, w2b_ref[ky],
                              preferred_element_type=jnp.float32)
    y2 = jnp.maximum(acc2 + b2_ref[...], 0.0).reshape(tb, 28, 896)

    mh = jnp.maximum(y2[:, 0:27, :], y2[:, 1:28, :])
    out_ref[...] = jnp.maximum(mh[:, :, 0:864],
                               mh[:, :, 32:896]).astype(jnp.bfloat16)


def _fc_kernel(x_ref, w1_ref, b1_ref, w2_ref, b2_ref, out_ref):
    h = jnp.dot(x_ref[...], w1_ref[...],
                preferred_element_type=jnp.float32) + b1_ref[...]
    out_ref[...] = jnp.dot(h, w2_ref[...],
                           preferred_element_type=jnp.float32) + b2_ref[...]


def _conv_stack(x2d, band1, b1t, band2, b2t, tb):
    bp = x2d.shape[0]
    return pl.pallas_call(
        _conv_kernel,
        out_shape=jax.ShapeDtypeStruct((bp, 27, 864), jnp.bfloat16),
        grid_spec=pltpu.PrefetchScalarGridSpec(
            num_scalar_prefetch=0,
            grid=(bp // tb,),
            in_specs=[
                pl.BlockSpec((tb, 28, 28), lambda b: (b, 0, 0)),
                pl.BlockSpec((3, 28, 896), lambda b: (0, 0, 0)),
                pl.BlockSpec((1, 896), lambda b: (0, 0)),
                pl.BlockSpec((3, 896, 896), lambda b: (0, 0, 0)),
                pl.BlockSpec((1, 896), lambda b: (0, 0)),
            ],
            out_specs=pl.BlockSpec((tb, 27, 864), lambda b: (b, 0, 0)),
            scratch_shapes=[
                pltpu.VMEM((tb, 30, 28), jnp.bfloat16),
                pltpu.VMEM((tb, 30, 896), jnp.bfloat16),
            ],
        ),
        compiler_params=pltpu.CompilerParams(
            dimension_semantics=("parallel",),
            vmem_limit_bytes=_VMEM_LIMIT_BYTES,
        ),
    )(x2d, band1, b1t, band2, b2t)


def _fc_stack(x_flat, w1, b1, w2, b2, tb):
    bp, k = x_flat.shape
    return pl.pallas_call(
        _fc_kernel,
        out_shape=jax.ShapeDtypeStruct((bp, 128), jnp.float32),
        grid_spec=pltpu.PrefetchScalarGridSpec(
            num_scalar_prefetch=0,
            grid=(bp // tb,),
            in_specs=[
                pl.BlockSpec((tb, k), lambda b: (b, 0)),
                pl.BlockSpec((k, 128), lambda b: (0, 0)),
                pl.BlockSpec((1, 128), lambda b: (0, 0)),
                pl.BlockSpec((128, 128), lambda b: (0, 0)),
                pl.BlockSpec((1, 128), lambda b: (0, 0)),
            ],
            out_specs=pl.BlockSpec((tb, 128), lambda b: (b, 0)),
        ),
        compiler_params=pltpu.CompilerParams(
            dimension_semantics=("parallel",),
            vmem_limit_bytes=_VMEM_LIMIT_BYTES,
        ),
    )(x_flat, w1, b1, w2, b2)


def kernel(x, band1, b1t, band2, b2t, wfc1_big, bfc1_r, wfc2_p, bfc2_p):
    b = x.shape[0]
    tb = min(b, 8)
    bp = ((b + tb - 1) // tb) * tb

    x2d = x[:, 0, :, :].astype(jnp.bfloat16)
    if bp != b:
        x2d = jnp.pad(x2d, ((0, bp - b), (0, 0), (0, 0)))

    band1h = band1.astype(jnp.bfloat16)
    band2h = band2.astype(jnp.bfloat16)
    wfc1h = wfc1_big.astype(jnp.bfloat16)

    pooled = _conv_stack(x2d, band1h, b1t, band2h, b2t, tb)
    x_flat = pooled.reshape(bp, 27 * 864)

    tb_fc = min(bp, 256)
    logits = _fc_stack(x_flat, wfc1h, bfc1_r, wfc2_p, bfc2_p, tb_fc)
    return logits[:b, :_NUM_CLASSES]


# speedup vs baseline: 1.4203x; 1.2086x over previous
import jax
import jax.numpy as jnp
from jax.experimental import pallas as pl
from jax.experimental.pallas import tpu as pltpu

_NUM_CLASSES = 10
_VMEM_LIMIT_BYTES = 48 * 1024 * 1024


def _fused_kernel(x_ref, w1b_ref, b1_ref, w2b_ref, b2_ref, wfc1_ref,
                  bfc1_ref, wfc2_ref, bfc2_ref, out_ref,
                  xpad_ref, y1pad_ref, slab_ref):
    tb = x_ref.shape[0]

    xpad_ref[...] = jnp.zeros_like(xpad_ref)
    xpad_ref[:, 1:29, :] = x_ref[...]

    acc1 = jnp.zeros((tb * 28, 896), jnp.float32)
    for ky in range(3):
        lhs = xpad_ref[:, ky:ky + 28, :].reshape(tb * 28, 28)
        acc1 = acc1 + jnp.dot(lhs, w1b_ref[ky],
                              preferred_element_type=jnp.float32)
    y1 = jnp.maximum(acc1 + b1_ref[...], 0.0)

    y1pad_ref[...] = jnp.zeros_like(y1pad_ref)
    y1pad_ref[:, 1:29, :] = y1.reshape(tb, 28, 896)

    acc2 = jnp.zeros((tb * 28, 896), jnp.float32)
    for ky in range(3):
        lhs = y1pad_ref[:, ky:ky + 28, :].reshape(tb * 28, 896)
        acc2 = acc2 + jnp.dot(lhs, w2b_ref[ky],
                              preferred_element_type=jnp.float32)
    y2 = jnp.maximum(acc2 + b2_ref[...], 0.0).reshape(tb, 28, 896)

    mh = jnp.maximum(y2[:, 0:27, :], y2[:, 1:28, :])
    slab_ref[...] = jnp.maximum(mh[:, :, 0:864], mh[:, :, 32:896])

    h = jnp.broadcast_to(bfc1_ref[...], (tb, 128)).astype(jnp.float32)
    for i in range(27):
        h = h + jnp.dot(slab_ref[:, i, :], wfc1_ref[i],
                        preferred_element_type=jnp.float32)
    out_ref[...] = jnp.dot(h, wfc2_ref[...],
                           preferred_element_type=jnp.float32) + bfc2_ref[...]


def _fused_forward(x2d, band1, b1t, band2, b2t, wfc1_r, bfc1_r, wfc2_p,
                   bfc2_p, tb):
    bp = x2d.shape[0]
    return pl.pallas_call(
        _fused_kernel,
        out_shape=jax.ShapeDtypeStruct((bp, 128), jnp.float32),
        grid_spec=pltpu.PrefetchScalarGridSpec(
            num_scalar_prefetch=0,
            grid=(bp // tb,),
            in_specs=[
                pl.BlockSpec((tb, 28, 28), lambda b: (b, 0, 0)),
                pl.BlockSpec((3, 28, 896), lambda b: (0, 0, 0)),
                pl.BlockSpec((1, 896), lambda b: (0, 0)),
                pl.BlockSpec((3, 896, 896), lambda b: (0, 0, 0)),
                pl.BlockSpec((1, 896), lambda b: (0, 0)),
                pl.BlockSpec((27, 864, 128), lambda b: (0, 0, 0)),
                pl.BlockSpec((1, 128), lambda b: (0, 0)),
                pl.BlockSpec((128, 128), lambda b: (0, 0)),
                pl.BlockSpec((1, 128), lambda b: (0, 0)),
            ],
            out_specs=pl.BlockSpec((tb, 128), lambda b: (b, 0)),
            scratch_shapes=[
                pltpu.VMEM((tb, 30, 28), jnp.float32),
                pltpu.VMEM((tb, 30, 896), jnp.float32),
                pltpu.VMEM((tb, 27, 864), jnp.float32),
            ],
        ),
        compiler_params=pltpu.CompilerParams(
            dimension_semantics=("parallel",),
            vmem_limit_bytes=_VMEM_LIMIT_BYTES,
        ),
    )(x2d, band1, b1t, band2, b2t, wfc1_r, bfc1_r, wfc2_p, bfc2_p)


def kernel(x, band1, b1t, band2, b2t, wfc1_big, bfc1_r, wfc2_p, bfc2_p):
    b = x.shape[0]
    tb = min(b, 8)
    bp = ((b + tb - 1) // tb) * tb

    x2d = x[:, 0, :, :]
    if bp != b:
        x2d = jnp.pad(x2d, ((0, bp - b), (0, 0), (0, 0)))

    wfc1_r = wfc1_big.reshape(27, 864, 128)
    logits = _fused_forward(x2d, band1, b1t, band2, b2t, wfc1_r, bfc1_r,
                            wfc2_p, bfc2_p, tb)
    return logits[:b, :_NUM_CLASSES]


# speedup vs baseline: 2.0084x; 1.4141x over previous
import jax
import jax.numpy as jnp
from jax.experimental import pallas as pl
from jax.experimental.pallas import tpu as pltpu

_NUM_CLASSES = 10
_VMEM_LIMIT_BYTES = 48 * 1024 * 1024


def _fused_kernel(x_ref, w1b_ref, b1_ref, w2b_ref, b2_ref, wfc1_ref,
                  bfc1_ref, wfc2_ref, bfc2_ref, out_ref,
                  w2bf_ref, xpad_ref, y1cat_ref, slab_ref):
    tb = x_ref.shape[0]
    m = tb * 32

    @pl.when(pl.program_id(0) == 0)
    def _prep():
        w2bf_ref[...] = w2b_ref[...].astype(jnp.bfloat16)
        xpad_ref[...] = jnp.zeros_like(xpad_ref)
        y1cat_ref[...] = jnp.zeros_like(y1cat_ref)

    xv = jnp.concatenate(
        [x_ref[...], jnp.zeros((tb, 4, 28), jnp.float32)], axis=1)
    xpad_ref[1:m + 1] = xv.reshape(m, 28)

    lhs1 = jnp.concatenate(
        [xpad_ref[ky:ky + m] for ky in range(3)], axis=1)
    acc1 = jnp.dot(lhs1, w1b_ref[...].reshape(84, 896),
                   preferred_element_type=jnp.float32)
    y1 = jnp.maximum(acc1 + b1_ref[...], 0.0)
    row = jax.lax.broadcasted_iota(jnp.int32, (m, 896), 0)
    y1 = jnp.where((row % 32) < 28, y1, 0.0).astype(jnp.bfloat16)

    y1cat_ref[1:m + 1, 0:896] = y1
    y1cat_ref[0:m, 896:1792] = y1
    y1cat_ref[0:m - 1, 1792:2688] = y1[1:m]
    acc2 = jnp.dot(y1cat_ref[0:m], w2bf_ref[...],
                   preferred_element_type=jnp.float32)
    y2v = jnp.maximum(acc2 + b2_ref[...], 0.0)

    mh = jnp.maximum(y2v, jnp.concatenate([y2v[1:m], y2v[0:1]], axis=0))
    slab_ref[...] = jnp.maximum(
        mh[:, 0:864], mh[:, 32:896]).reshape(tb, 32, 864)

    h = jnp.broadcast_to(bfc1_ref[...], (tb, 128)).astype(jnp.float32)
    for i in range(27):
        h = h + jnp.dot(slab_ref[:, i, :], wfc1_ref[i],
                        preferred_element_type=jnp.float32)
    out_ref[...] = jnp.dot(h, wfc2_ref[...],
                           preferred_element_type=jnp.float32) + bfc2_ref[...]


def _fused_forward(x2d, band1, b1t, band2, b2t, wfc1_r, bfc1_r, wfc2_p,
                   bfc2_p, tb):
    bp = x2d.shape[0]
    m = tb * 32
    return pl.pallas_call(
        _fused_kernel,
        out_shape=jax.ShapeDtypeStruct((bp, 128), jnp.float32),
        grid_spec=pltpu.PrefetchScalarGridSpec(
            num_scalar_prefetch=0,
            grid=(bp // tb,),
            in_specs=[
                pl.BlockSpec((tb, 28, 28), lambda b: (b, 0, 0)),
                pl.BlockSpec((3, 28, 896), lambda b: (0, 0, 0)),
                pl.BlockSpec((1, 896), lambda b: (0, 0)),
                pl.BlockSpec((2688, 896), lambda b: (0, 0)),
                pl.BlockSpec((1, 896), lambda b: (0, 0)),
                pl.BlockSpec((27, 864, 128), lambda b: (0, 0, 0)),
                pl.BlockSpec((1, 128), lambda b: (0, 0)),
                pl.BlockSpec((128, 128), lambda b: (0, 0)),
                pl.BlockSpec((1, 128), lambda b: (0, 0)),
            ],
            out_specs=pl.BlockSpec((tb, 128), lambda b: (b, 0)),
            scratch_shapes=[
                pltpu.VMEM((2688, 896), jnp.bfloat16),
                pltpu.VMEM((tb * 32 + 8, 28), jnp.float32),
                pltpu.VMEM((tb * 32 + 8, 2688), jnp.bfloat16),
                pltpu.VMEM((tb, 32, 864), jnp.float32),
            ],
        ),
        compiler_params=pltpu.CompilerParams(
            dimension_semantics=("arbitrary",),
            vmem_limit_bytes=_VMEM_LIMIT_BYTES,
        ),
    )(x2d, band1, b1t, band2, b2t, wfc1_r, bfc1_r, wfc2_p, bfc2_p)


def kernel(x, band1, b1t, band2, b2t, wfc1_big, bfc1_r, wfc2_p, bfc2_p):
    b = x.shape[0]
    tb = min(b, 32)
    bp = ((b + tb - 1) // tb) * tb

    x2d = x[:, 0, :, :]
    if bp != b:
        x2d = jnp.pad(x2d, ((0, bp - b), (0, 0), (0, 0)))

    wfc1_r = wfc1_big.reshape(27, 864, 128)
    band2_cat = band2.reshape(2688, 896)
    logits = _fused_forward(x2d, band1, b1t, band2_cat, b2t, wfc1_r, bfc1_r,
                            wfc2_p, bfc2_p, tb)
    return logits[:b, :_NUM_CLASSES]


# speedup vs baseline: 2.0766x; 1.0340x over previous
import jax
import jax.numpy as jnp
from jax.experimental import pallas as pl
from jax.experimental.pallas import tpu as pltpu

_NUM_CLASSES = 10
_VMEM_LIMIT_BYTES = 48 * 1024 * 1024


def _fused_kernel(x_ref, w1b_ref, b1_ref, w2b_ref, b2_ref, wfc1_ref,
                  bfc1_ref, wfc2_ref, bfc2_ref, out_ref,
                  b32_ref, w2bf_ref, xpad_ref, y1cat_ref, slab_ref):
    tb = x_ref.shape[0]
    m = tb * 32
    ht = tb // 2
    hm = ht * 32

    @pl.when(pl.program_id(0) == 0)
    def _prep():
        r32 = jax.lax.broadcasted_iota(jnp.int32, (32, 896), 0)
        b32_ref[...] = jnp.where(
            r32 < 28, jnp.broadcast_to(b1_ref[...], (32, 896)), -1e30)
        w2bf_ref[...] = w2b_ref[...].astype(jnp.bfloat16)
        xpad_ref[...] = jnp.zeros_like(xpad_ref)
        y1cat_ref[...] = jnp.zeros_like(y1cat_ref)

    w1c = w1b_ref[...].reshape(84, 896)
    xall = x_ref[...].reshape(tb, 28, 28)

    for s in range(2):
        r0 = s * hm
        xv = jnp.concatenate(
            [xall[s * ht:(s + 1) * ht],
             jnp.zeros((ht, 4, 28), jnp.float32)], axis=1)
        xpad_ref[r0 + 1:r0 + hm + 1] = xv.reshape(hm, 28)

        lhs1 = jnp.concatenate(
            [xpad_ref[r0 + ky:r0 + ky + hm] for ky in range(3)], axis=1)
        acc1 = jnp.dot(lhs1, w1c, preferred_element_type=jnp.float32)
        y1 = jnp.maximum(
            acc1.reshape(ht, 32, 896) + b32_ref[...], 0.0
        ).reshape(hm, 896).astype(jnp.bfloat16)

        y1cat_ref[r0 + 1:r0 + hm + 1, 0:896] = y1
        y1cat_ref[r0:r0 + hm, 896:1792] = y1
        y1cat_ref[r0:r0 + hm - 1, 1792:2688] = y1[1:hm]
        acc2 = jnp.dot(y1cat_ref[r0:r0 + hm], w2bf_ref[...],
                       preferred_element_type=jnp.float32)
        y2v = jnp.maximum(acc2 + b2_ref[...], 0.0)

        mh = jnp.maximum(y2v, jnp.concatenate([y2v[1:hm], y2v[0:1]], axis=0))
        slab_ref[s * ht:(s + 1) * ht] = jnp.maximum(
            mh[:, 0:864], mh[:, 32:896]).reshape(ht, 32, 864)

    parts = []
    for j in range(4):
        hj = jnp.zeros((tb, 128), jnp.float32)
        for i in range(j, 27, 4):
            hj = hj + jnp.dot(slab_ref[:, i, :], wfc1_ref[i],
                              preferred_element_type=jnp.float32)
        parts.append(hj)
    h = ((parts[0] + parts[1]) + (parts[2] + parts[3])
         + jnp.broadcast_to(bfc1_ref[...], (tb, 128)))
    out_ref[...] = jnp.dot(h, wfc2_ref[...],
                           preferred_element_type=jnp.float32) + bfc2_ref[...]


def _fused_forward(x4d, band1, b1t, band2_cat, b2t, wfc1_r, bfc1_r, wfc2_p,
                   bfc2_p, tb):
    bp = x4d.shape[0]
    return pl.pallas_call(
        _fused_kernel,
        out_shape=jax.ShapeDtypeStruct((bp, 128), jnp.float32),
        grid_spec=pltpu.PrefetchScalarGridSpec(
            num_scalar_prefetch=0,
            grid=(bp // tb,),
            in_specs=[
                pl.BlockSpec((tb, 1, 28, 28), lambda b: (b, 0, 0, 0)),
                pl.BlockSpec((3, 28, 896), lambda b: (0, 0, 0)),
                pl.BlockSpec((1, 896), lambda b: (0, 0)),
                pl.BlockSpec((2688, 896), lambda b: (0, 0)),
                pl.BlockSpec((1, 896), lambda b: (0, 0)),
                pl.BlockSpec((27, 864, 128), lambda b: (0, 0, 0)),
                pl.BlockSpec((1, 128), lambda b: (0, 0)),
                pl.BlockSpec((128, 128), lambda b: (0, 0)),
                pl.BlockSpec((1, 128), lambda b: (0, 0)),
            ],
            out_specs=pl.BlockSpec((tb, 128), lambda b: (b, 0)),
            scratch_shapes=[
                pltpu.VMEM((32, 896), jnp.float32),
                pltpu.VMEM((2688, 896), jnp.bfloat16),
                pltpu.VMEM((tb * 32 + 8, 28), jnp.float32),
                pltpu.VMEM((tb * 32 + 8, 2688), jnp.bfloat16),
                pltpu.VMEM((tb, 32, 864), jnp.float32),
            ],
        ),
        compiler_params=pltpu.CompilerParams(
            dimension_semantics=("arbitrary",),
            vmem_limit_bytes=_VMEM_LIMIT_BYTES,
        ),
    )(x4d, band1, b1t, band2_cat, b2t, wfc1_r, bfc1_r, wfc2_p, bfc2_p)


def kernel(x, band1, b1t, band2, b2t, wfc1_big, bfc1_r, wfc2_p, bfc2_p):
    b = x.shape[0]
    tb = min(b, 32)
    bp = ((b + tb - 1) // tb) * tb

    x2d = x
    if bp != b:
        x2d = jnp.pad(x2d, ((0, bp - b), (0, 0), (0, 0), (0, 0)))

    wfc1_r = wfc1_big.reshape(27, 864, 128)
    band2_cat = band2.reshape(2688, 896)
    logits = _fused_forward(x2d, band1, b1t, band2_cat, b2t, wfc1_r, bfc1_r,
                            wfc2_p, bfc2_p, tb)
    return logits[:b, :_NUM_CLASSES]


# speedup vs baseline: 2.0774x; 1.0004x over previous
import jax
import jax.numpy as jnp
from jax.experimental import pallas as pl
from jax.experimental.pallas import tpu as pltpu

_NUM_CLASSES = 10
_VMEM_LIMIT_BYTES = 48 * 1024 * 1024


def _fused_kernel(x_ref, w1b_ref, b1_ref, w2b_ref, b2_ref, wfc1_ref,
                  bfc1_ref, wfc2_ref, bfc2_ref, out_ref,
                  b32_ref, w2bf_ref, xpad_ref, y1cat_ref, slab_ref):
    tb = x_ref.shape[0]
    m = tb * 32
    ht = tb // 2
    hm = ht * 32

    @pl.when(pl.program_id(0) == 0)
    def _prep():
        r32 = jax.lax.broadcasted_iota(jnp.int32, (32, 896), 0)
        b32_ref[...] = jnp.where(
            r32 < 28, jnp.broadcast_to(b1_ref[...], (32, 896)), -1e30)
        w2bf_ref[...] = w2b_ref[...].astype(jnp.bfloat16)
        xpad_ref[...] = jnp.zeros_like(xpad_ref)
        y1cat_ref[...] = jnp.zeros_like(y1cat_ref)

    w1c = w1b_ref[...].reshape(84, 896)
    xall = x_ref[...].reshape(tb, 28, 28)

    for s in range(2):
        r0 = s * hm
        xv = jnp.concatenate(
            [xall[s * ht:(s + 1) * ht],
             jnp.zeros((ht, 4, 28), jnp.float32)], axis=1)
        xpad_ref[r0 + 1:r0 + hm + 1] = xv.reshape(hm, 28)

        lhs1 = jnp.concatenate(
            [xpad_ref[r0 + ky:r0 + ky + hm] for ky in range(3)], axis=1)
        acc1 = jnp.dot(lhs1, w1c, preferred_element_type=jnp.float32)
        y1 = jnp.maximum(
            acc1.reshape(ht, 32, 896) + b32_ref[...], 0.0
        ).reshape(hm, 896).astype(jnp.bfloat16)

        y1cat_ref[r0 + 1:r0 + hm + 1, 0:896] = y1
        y1cat_ref[r0:r0 + hm, 896:1792] = y1
        y1cat_ref[r0:r0 + hm - 1, 1792:2688] = y1[1:hm]
        acc2 = jnp.dot(y1cat_ref[r0:r0 + hm], w2bf_ref[...],
                       preferred_element_type=jnp.float32)
        y2v = jnp.maximum(acc2 + b2_ref[...], 0.0)

        mh = jnp.maximum(y2v, jnp.concatenate([y2v[1:hm], y2v[0:1]], axis=0))
        slab_ref[s * ht:(s + 1) * ht] = jnp.maximum(
            mh[:, 0:864], mh[:, 32:896]).reshape(ht, 32, 864)

    parts = []
    for j in range(4):
        hj = jnp.zeros((tb, 128), jnp.float32)
        for i in range(j, 27, 4):
            hj = hj + jnp.dot(slab_ref[:, i, :], wfc1_ref[i],
                              preferred_element_type=jnp.float32)
        parts.append(hj)
    h = ((parts[0] + parts[1]) + (parts[2] + parts[3])
         + jnp.broadcast_to(bfc1_ref[...], (tb, 128)))
    out_ref[...] = jnp.dot(h, wfc2_ref[...],
                           preferred_element_type=jnp.float32) + bfc2_ref[...]


def _fused_forward(x4d, band1, b1t, band2_cat, b2t, wfc1_r, bfc1_r, wfc2_p,
                   bfc2_p, tb):
    bp = x4d.shape[0]
    return pl.pallas_call(
        _fused_kernel,
        out_shape=jax.ShapeDtypeStruct((bp, 128), jnp.float32),
        grid_spec=pltpu.PrefetchScalarGridSpec(
            num_scalar_prefetch=0,
            grid=(bp // tb,),
            in_specs=[
                pl.BlockSpec((tb, 1, 28, 28), lambda b: (b, 0, 0, 0)),
                pl.BlockSpec((3, 28, 896), lambda b: (0, 0, 0)),
                pl.BlockSpec((1, 896), lambda b: (0, 0)),
                pl.BlockSpec((2688, 896), lambda b: (0, 0)),
                pl.BlockSpec((1, 896), lambda b: (0, 0)),
                pl.BlockSpec((27, 864, 128), lambda b: (0, 0, 0)),
                pl.BlockSpec((1, 128), lambda b: (0, 0)),
                pl.BlockSpec((128, 128), lambda b: (0, 0)),
                pl.BlockSpec((1, 128), lambda b: (0, 0)),
            ],
            out_specs=pl.BlockSpec((tb, 128), lambda b: (b, 0)),
            scratch_shapes=[
                pltpu.VMEM((32, 896), jnp.float32),
                pltpu.VMEM((2688, 896), jnp.bfloat16),
                pltpu.VMEM((tb * 32 + 8, 28), jnp.float32),
                pltpu.VMEM((tb * 32 + 8, 2688), jnp.bfloat16),
                pltpu.VMEM((tb, 32, 864), jnp.float32),
            ],
        ),
        compiler_params=pltpu.CompilerParams(
            dimension_semantics=("arbitrary",),
            vmem_limit_bytes=_VMEM_LIMIT_BYTES,
        ),
    )(x4d, band1, b1t, band2_cat, b2t, wfc1_r, bfc1_r, wfc2_p, bfc2_p)


def kernel(x, band1, b1t, band2, b2t, wfc1_big, bfc1_r, wfc2_p, bfc2_p):
    b = x.shape[0]
    tb = min(b, 32)
    tb = tb + (tb & 1)
    bp = ((b + tb - 1) // tb) * tb

    x2d = x
    if bp != b:
        x2d = jnp.pad(x2d, ((0, bp - b), (0, 0), (0, 0), (0, 0)))

    wfc1_r = wfc1_big.reshape(27, 864, 128)
    band2_cat = band2.reshape(2688, 896)
    logits = _fused_forward(x2d, band1, b1t, band2_cat, b2t, wfc1_r, bfc1_r,
                            wfc2_p, bfc2_p, tb)
    return logits[:b, :_NUM_CLASSES]


# speedup vs baseline: 2.1918x; 1.0551x over previous
import jax
import jax.numpy as jnp
from jax.experimental import pallas as pl
from jax.experimental.pallas import tpu as pltpu

_NUM_CLASSES = 10
_VMEM_LIMIT_BYTES = 56 * 1024 * 1024


def _fused_kernel(x_ref, w1b_ref, b1_ref, w2b_ref, b2_ref, wfc1_ref,
                  bfc1_ref, wfc2_ref, bfc2_ref, out_ref,
                  b32_ref, w2bf_ref, xpad_ref, y1cat_ref, slab_ref):
    tb = x_ref.shape[0]
    m = tb * 32
    ht = tb // 2
    hm = ht * 32

    @pl.when(pl.program_id(0) == 0)
    def _prep():
        r32 = jax.lax.broadcasted_iota(jnp.int32, (32, 896), 0)
        b32_ref[...] = jnp.where(
            r32 < 28, jnp.broadcast_to(b1_ref[...], (32, 896)), -1e30)
        w2bf_ref[...] = w2b_ref[...].astype(jnp.bfloat16)
        xpad_ref[...] = jnp.zeros_like(xpad_ref)
        y1cat_ref[...] = jnp.zeros_like(y1cat_ref)

    w1c = w1b_ref[...].reshape(84, 896)
    xall = x_ref[...].reshape(tb, 28, 28)

    for s in range(2):
        r0 = s * hm
        xv = jnp.concatenate(
            [xall[s * ht:(s + 1) * ht],
             jnp.zeros((ht, 4, 28), jnp.float32)], axis=1)
        xpad_ref[r0 + 1:r0 + hm + 1] = xv.reshape(hm, 28)

        lhs1 = jnp.concatenate(
            [xpad_ref[r0 + ky:r0 + ky + hm] for ky in range(3)], axis=1)
        acc1 = jnp.dot(lhs1, w1c, preferred_element_type=jnp.float32)
        y1 = jnp.maximum(
            acc1.reshape(ht, 32, 896) + b32_ref[...], 0.0
        ).reshape(hm, 896).astype(jnp.bfloat16)

        y1cat_ref[r0 + 1:r0 + hm + 1, 0:896] = y1
        y1cat_ref[r0:r0 + hm, 896:1792] = y1
        y1cat_ref[r0:r0 + hm - 1, 1792:2688] = y1[1:hm]
        acc2 = jnp.dot(y1cat_ref[r0:r0 + hm], w2bf_ref[...],
                       preferred_element_type=jnp.float32)
        y2v = jnp.maximum(acc2 + b2_ref[...], 0.0)

        mh = jnp.maximum(y2v, jnp.concatenate([y2v[1:hm], y2v[0:1]], axis=0))
        slab_ref[s * ht:(s + 1) * ht] = jnp.maximum(
            mh[:, 0:864], mh[:, 32:896]).reshape(ht, 32, 864)

    parts = []
    for j in range(4):
        hj = jnp.zeros((tb, 128), jnp.float32)
        for i in range(j, 27, 4):
            hj = hj + jnp.dot(slab_ref[:, i, :], wfc1_ref[i],
                              preferred_element_type=jnp.float32)
        parts.append(hj)
    h = ((parts[0] + parts[1]) + (parts[2] + parts[3])
         + jnp.broadcast_to(bfc1_ref[...], (tb, 128)))
    out_ref[...] = jnp.dot(h, wfc2_ref[...],
                           preferred_element_type=jnp.float32) + bfc2_ref[...]


def _fused_forward(x4d, band1, b1t, band2_cat, b2t, wfc1_r, bfc1_r, wfc2_p,
                   bfc2_p, tb):
    bp = x4d.shape[0]
    return pl.pallas_call(
        _fused_kernel,
        out_shape=jax.ShapeDtypeStruct((bp, 128), jnp.float32),
        grid_spec=pltpu.PrefetchScalarGridSpec(
            num_scalar_prefetch=0,
            grid=(bp // tb,),
            in_specs=[
                pl.BlockSpec((tb, 1, 28, 28), lambda b: (b, 0, 0, 0)),
                pl.BlockSpec((3, 28, 896), lambda b: (0, 0, 0)),
                pl.BlockSpec((1, 896), lambda b: (0, 0)),
                pl.BlockSpec((2688, 896), lambda b: (0, 0)),
                pl.BlockSpec((1, 896), lambda b: (0, 0)),
                pl.BlockSpec((27, 864, 128), lambda b: (0, 0, 0)),
                pl.BlockSpec((1, 128), lambda b: (0, 0)),
                pl.BlockSpec((128, 128), lambda b: (0, 0)),
                pl.BlockSpec((1, 128), lambda b: (0, 0)),
            ],
            out_specs=pl.BlockSpec((tb, 128), lambda b: (b, 0)),
            scratch_shapes=[
                pltpu.VMEM((32, 896), jnp.float32),
                pltpu.VMEM((2688, 896), jnp.bfloat16),
                pltpu.VMEM((tb * 32 + 8, 28), jnp.float32),
                pltpu.VMEM((tb * 32 + 8, 2688), jnp.bfloat16),
                pltpu.VMEM((tb, 32, 864), jnp.float32),
            ],
        ),
        compiler_params=pltpu.CompilerParams(
            dimension_semantics=("arbitrary",),
            vmem_limit_bytes=_VMEM_LIMIT_BYTES,
        ),
    )(x4d, band1, b1t, band2_cat, b2t, wfc1_r, bfc1_r, wfc2_p, bfc2_p)


def kernel(x, band1, b1t, band2, b2t, wfc1_big, bfc1_r, wfc2_p, bfc2_p):
    b = x.shape[0]
    tb = min(b, 64)
    tb = tb + (tb & 1)
    bp = ((b + tb - 1) // tb) * tb

    x2d = x
    if bp != b:
        x2d = jnp.pad(x2d, ((0, bp - b), (0, 0), (0, 0), (0, 0)))

    wfc1_r = wfc1_big.reshape(27, 864, 128)
    band2_cat = band2.reshape(2688, 896)
    logits = _fused_forward(x2d, band1, b1t, band2_cat, b2t, wfc1_r, bfc1_r,
                            wfc2_p, bfc2_p, tb)
    return logits[:b, :_NUM_CLASSES]


# speedup vs baseline: 2.2454x; 1.0245x over previous
import jax
import jax.numpy as jnp
from jax.experimental import pallas as pl
from jax.experimental.pallas import tpu as pltpu

_NUM_CLASSES = 10
_VMEM_LIMIT_BYTES = 56 * 1024 * 1024


def _fused_kernel(x_ref, w1b_ref, b1_ref, w2b_ref, b2_ref, wfc1_ref,
                  bfc1_ref, wfc2_ref, bfc2_ref, out_ref,
                  b32_ref, w2bf_ref, xpad_ref, y1cat_ref, slab_ref):
    tb = x_ref.shape[0]
    m = tb * 32
    ht = tb // 4
    hm = ht * 32

    @pl.when(pl.program_id(0) == 0)
    def _prep():
        r32 = jax.lax.broadcasted_iota(jnp.int32, (32, 896), 0)
        b32_ref[...] = jnp.where(
            r32 < 28, jnp.broadcast_to(b1_ref[...], (32, 896)), -1e30)
        w2bf_ref[...] = w2b_ref[...].astype(jnp.bfloat16)
        xpad_ref[...] = jnp.zeros_like(xpad_ref)
        y1cat_ref[...] = jnp.zeros_like(y1cat_ref)

    w1c = w1b_ref[...].reshape(84, 896)
    xall = x_ref[...].reshape(tb, 28, 28)

    for s in range(4):
        r0 = s * hm
        xv = jnp.concatenate(
            [xall[s * ht:(s + 1) * ht],
             jnp.zeros((ht, 4, 28), jnp.float32)], axis=1)
        xpad_ref[r0 + 1:r0 + hm + 1] = xv.reshape(hm, 28)

        lhs1 = jnp.concatenate(
            [xpad_ref[r0 + ky:r0 + ky + hm] for ky in range(3)], axis=1)
        acc1 = jnp.dot(lhs1, w1c, preferred_element_type=jnp.float32)
        y1 = jnp.maximum(
            acc1.reshape(ht, 32, 896) + b32_ref[...], 0.0
        ).reshape(hm, 896).astype(jnp.bfloat16)

        y1cat_ref[r0 + 1:r0 + hm + 1, 0:896] = y1
        y1cat_ref[r0:r0 + hm, 896:1792] = y1
        y1cat_ref[r0:r0 + hm - 1, 1792:2688] = y1[1:hm]
        acc2 = jnp.dot(y1cat_ref[r0:r0 + hm], w2bf_ref[...],
                       preferred_element_type=jnp.float32)
        y2v = jnp.maximum(acc2 + b2_ref[...], 0.0)

        mh = jnp.maximum(y2v, jnp.concatenate([y2v[1:hm], y2v[0:1]], axis=0))
        slab_ref[s * ht:(s + 1) * ht] = jnp.maximum(
            mh[:, 0:864], mh[:, 32:896]).reshape(ht, 32, 864)

    parts = []
    for j in range(4):
        hj = jnp.zeros((tb, 128), jnp.float32)
        for i in range(j, 27, 4):
            hj = hj + jnp.dot(slab_ref[:, i, :], wfc1_ref[i],
                              preferred_element_type=jnp.float32)
        parts.append(hj)
    h = ((parts[0] + parts[1]) + (parts[2] + parts[3])
         + jnp.broadcast_to(bfc1_ref[...], (tb, 128)))
    out_ref[...] = jnp.dot(h, wfc2_ref[...],
                           preferred_element_type=jnp.float32) + bfc2_ref[...]


def _fused_forward(x4d, band1, b1t, band2_cat, b2t, wfc1_r, bfc1_r, wfc2_p,
                   bfc2_p, tb):
    bp = x4d.shape[0]
    return pl.pallas_call(
        _fused_kernel,
        out_shape=jax.ShapeDtypeStruct((bp, 128), jnp.float32),
        grid_spec=pltpu.PrefetchScalarGridSpec(
            num_scalar_prefetch=0,
            grid=(bp // tb,),
            in_specs=[
                pl.BlockSpec((tb, 1, 28, 28), lambda b: (b, 0, 0, 0)),
                pl.BlockSpec((3, 28, 896), lambda b: (0, 0, 0)),
                pl.BlockSpec((1, 896), lambda b: (0, 0)),
                pl.BlockSpec((2688, 896), lambda b: (0, 0)),
                pl.BlockSpec((1, 896), lambda b: (0, 0)),
                pl.BlockSpec((27, 864, 128), lambda b: (0, 0, 0)),
                pl.BlockSpec((1, 128), lambda b: (0, 0)),
                pl.BlockSpec((128, 128), lambda b: (0, 0)),
                pl.BlockSpec((1, 128), lambda b: (0, 0)),
            ],
            out_specs=pl.BlockSpec((tb, 128), lambda b: (b, 0)),
            scratch_shapes=[
                pltpu.VMEM((32, 896), jnp.float32),
                pltpu.VMEM((2688, 896), jnp.bfloat16),
                pltpu.VMEM((tb * 32 + 8, 28), jnp.float32),
                pltpu.VMEM((tb * 32 + 8, 2688), jnp.bfloat16),
                pltpu.VMEM((tb, 32, 864), jnp.float32),
            ],
        ),
        compiler_params=pltpu.CompilerParams(
            dimension_semantics=("arbitrary",),
            vmem_limit_bytes=_VMEM_LIMIT_BYTES,
        ),
    )(x4d, band1, b1t, band2_cat, b2t, wfc1_r, bfc1_r, wfc2_p, bfc2_p)


def kernel(x, band1, b1t, band2, b2t, wfc1_big, bfc1_r, wfc2_p, bfc2_p):
    b = x.shape[0]
    tb = min(b, 64)
    tb = ((tb + 3) // 4) * 4
    bp = ((b + tb - 1) // tb) * tb

    x2d = x
    if bp != b:
        x2d = jnp.pad(x2d, ((0, bp - b), (0, 0), (0, 0), (0, 0)))

    wfc1_r = wfc1_big.reshape(27, 864, 128)
    band2_cat = band2.reshape(2688, 896)
    logits = _fused_forward(x2d, band1, b1t, band2_cat, b2t, wfc1_r, bfc1_r,
                            wfc2_p, bfc2_p, tb)
    return logits[:b, :_NUM_CLASSES]
